# Initial kernel scaffold; baseline (speedup 1.0000x reference)
import functools

import jax
import jax.numpy as jnp
from jax import lax
from jax.experimental import pallas as pl
from jax.experimental.pallas import tpu as pltpu
from jax.experimental.pallas import tpu_sc as plsc

NC = 2
NS = 16
NW = NC * NS
CH = 128
GARB = 1136

_MESH = dict(core_axis_name="c", subcore_axis_name="s")


def _pad_edges(idx_val, idx_tgt, n):
    e = idx_val.shape[0]
    ep = ((e + NW * CH - 1) // (NW * CH)) * (NW * CH)
    pad = ep - e
    ar = jnp.arange(pad, dtype=jnp.int32)
    val_p = jnp.concatenate([idx_val, ar % n])
    tgt_p = jnp.concatenate([idx_tgt, n + (ar % GARB)])
    return val_p, tgt_p, ep


def _make_deg(n_acc, ep):
    ew = ep // NW
    cpt = ew // CH
    rpt = n_acc // NS
    mesh = plsc.VectorSubcoreMesh(**_MESH)

    @functools.partial(
        pl.kernel,
        mesh=mesh,
        out_type=jax.ShapeDtypeStruct((NC, n_acc), jnp.float32),
        scratch_types=[
            pltpu.VMEM((CH,), jnp.int32),
            pltpu.VMEM((CH,), jnp.float32),
            pltpu.VMEM_SHARED((n_acc,), jnp.float32),
        ],
    )
    def deg_k(dst_hbm, zero_hbm, out_hbm, idx_d, ones_v, deg_sh):
        c = lax.axis_index("c")
        s = lax.axis_index("s")
        wid = c * NS + s
        for j in range(CH // 16):
            ones_v[pl.ds(16 * j, 16)] = jnp.ones((16,), jnp.float32)
        r0 = s * rpt
        pltpu.sync_copy(zero_hbm.at[pl.ds(r0, rpt)], deg_sh.at[pl.ds(r0, rpt)])
        plsc.subcore_barrier()

        def body(i, carry):
            base = wid * ew + i * CH
            pltpu.sync_copy(dst_hbm.at[pl.ds(base, CH)], idx_d)
            pltpu.sync_copy(ones_v, deg_sh.at[idx_d], add=True)
            return carry

        lax.fori_loop(0, cpt, body, 0)
        plsc.subcore_barrier()
        pltpu.sync_copy(deg_sh.at[pl.ds(r0, rpt)], out_hbm.at[c, pl.ds(r0, rpt)])

    return deg_k


def _make_agg(n_acc, d, ep):
    ew = ep // NW
    cpt = ew // CH
    rpt = n_acc // NS
    mesh = plsc.VectorSubcoreMesh(**_MESH)

    @functools.partial(
        pl.kernel,
        mesh=mesh,
        out_type=jax.ShapeDtypeStruct((NC, n_acc, d), jnp.float32),
        scratch_types=[
            pltpu.VMEM((CH,), jnp.int32),
            pltpu.VMEM((CH,), jnp.int32),
            pltpu.VMEM((CH, d), jnp.float32),
            pltpu.VMEM_SHARED((n_acc, d), jnp.float32),
            pltpu.SemaphoreType.DMA,
        ],
    )
    def agg_k(u_hbm, src_hbm, dst_hbm, zero_hbm, out_hbm,
              idx_s, idx_d, rows, acc_sh, sem):
        c = lax.axis_index("c")
        s = lax.axis_index("s")
        wid = c * NS + s
        r0 = s * rpt
        pltpu.sync_copy(zero_hbm.at[pl.ds(r0, rpt)], acc_sh.at[pl.ds(r0, rpt)])
        plsc.subcore_barrier()

        def body(i, carry):
            base = wid * ew + i * CH
            pltpu.sync_copy(src_hbm.at[pl.ds(base, CH)], idx_s)
            pltpu.sync_copy(dst_hbm.at[pl.ds(base, CH)], idx_d)
            pltpu.async_copy(u_hbm.at[idx_s], rows, sem).wait()
            pltpu.sync_copy(rows, acc_sh.at[idx_d], add=True)
            return carry

        lax.fori_loop(0, cpt, body, 0)
        plsc.subcore_barrier()
        pltpu.sync_copy(acc_sh.at[pl.ds(r0, rpt)], out_hbm.at[c, pl.ds(r0, rpt)])

    return agg_k


def _make_dec(d, ep):
    ew = ep // NW
    cpt = ew // CH
    mesh = plsc.VectorSubcoreMesh(**_MESH)

    @functools.partial(
        pl.kernel,
        mesh=mesh,
        out_type=jax.ShapeDtypeStruct((ep * 16,), jnp.float32),
        scratch_types=[
            pltpu.VMEM((CH,), jnp.int32),
            pltpu.VMEM((CH,), jnp.int32),
            pltpu.VMEM((CH, d), jnp.float32),
            pltpu.VMEM((CH, d), jnp.float32),
            pltpu.VMEM((CH * 16,), jnp.float32),
            pltpu.SemaphoreType.DMA,
            pltpu.SemaphoreType.DMA,
        ],
    )
    def dec_k(z_hbm, a_hbm, b_hbm, out_hbm,
              idx_a, idx_b, za, zb, part, sema, semb):
        c = lax.axis_index("c")
        s = lax.axis_index("s")
        wid = c * NS + s

        def body(i, carry):
            base = wid * ew + i * CH
            pltpu.sync_copy(a_hbm.at[pl.ds(base, CH)], idx_a)
            pltpu.sync_copy(b_hbm.at[pl.ds(base, CH)], idx_b)
            ca = pltpu.async_copy(z_hbm.at[idx_a], za, sema)
            cb = pltpu.async_copy(z_hbm.at[idx_b], zb, semb)
            ca.wait()
            cb.wait()

            def edge(e2, carry2):
                acc = za[e2, pl.ds(0, 16)] * zb[e2, pl.ds(0, 16)]
                for j in range(1, d // 16):
                    acc = acc + za[e2, pl.ds(16 * j, 16)] * zb[e2, pl.ds(16 * j, 16)]
                part[pl.ds(e2 * 16, 16)] = acc
                return carry2

            lax.fori_loop(0, CH, edge, 0)
            pltpu.sync_copy(part, out_hbm.at[pl.ds(base * 16, CH * 16)])
            return carry

        lax.fori_loop(0, cpt, body, 0)

    return dec_k


def _tc_encode1(x, W1, deg_t):
    n, d_hid = x.shape[0], W1.shape[1]

    def body(x_ref, w_ref, deg_ref, u1_ref, dis_ref):
        deg = deg_ref[:, 0:1] + deg_ref[:, 1:2] + 1.0
        dis = lax.rsqrt(deg)
        xw = jnp.dot(x_ref[...], w_ref[...],
                     preferred_element_type=jnp.float32,
                     precision=lax.Precision.HIGHEST)
        u1_ref[...] = xw * dis
        dis_ref[...] = dis

    return pl.pallas_call(
        body,
        out_shape=(jax.ShapeDtypeStruct((n, d_hid), jnp.float32),
                   jax.ShapeDtypeStruct((n, 1), jnp.float32)),
    )(x, W1, deg_t)


def _tc_mid(pa, pb, u1, dis, b1, W2):
    n, d_out = u1.shape[0], W2.shape[1]

    def body(pa_ref, pb_ref, u1_ref, dis_ref, b1_ref, w2_ref, u2_ref):
        acc = pa_ref[...] + pb_ref[...] + u1_ref[...]
        h = jnp.maximum(dis_ref[...] * acc + b1_ref[...], 0.0)
        hw = jnp.dot(h, w2_ref[...],
                     preferred_element_type=jnp.float32,
                     precision=lax.Precision.HIGHEST)
        u2_ref[...] = hw * dis_ref[...]

    return pl.pallas_call(
        body,
        out_shape=jax.ShapeDtypeStruct((n, d_out), jnp.float32),
    )(pa, pb, u1, dis, b1, W2)


def _tc_final(pa, pb, u2, dis, b2):
    n, d_out = u2.shape

    def body(pa_ref, pb_ref, u2_ref, dis_ref, b2_ref, z_ref):
        acc = pa_ref[...] + pb_ref[...] + u2_ref[...]
        z_ref[...] = dis_ref[...] * acc + b2_ref[...]

    return pl.pallas_call(
        body,
        out_shape=jax.ShapeDtypeStruct((n, d_out), jnp.float32),
    )(pa, pb, u2, dis, b2)


def _tc_reduce16(p3):
    m = p3.shape[0]

    def body(p_ref, o_ref):
        o_ref[...] = jnp.sum(p_ref[...], axis=-1)

    return pl.pallas_call(
        body,
        out_shape=jax.ShapeDtypeStruct((m, 128), jnp.float32),
    )(p3)


@jax.jit
def kernel(x, edge_index, pos_edge_index, neg_edge_index, W1, b1, W2, b2):
    n = x.shape[0]
    d_hid = W1.shape[1]
    d_out = W2.shape[1]
    n_acc = n + GARB

    src, dst = edge_index[0], edge_index[1]
    src_p, dst_p, ep = _pad_edges(src, dst, n)

    ei = jnp.concatenate([pos_edge_index, neg_edge_index], axis=1)
    e_dec = ei.shape[1]
    a_p, b_p, ep_dec = _pad_edges(ei[0], ei[1], n)
    b_p = jnp.where(jnp.arange(ep_dec) < e_dec, b_p, b_p % n)

    zeros1 = jnp.zeros((n_acc,), jnp.float32)
    zeros128 = jnp.zeros((n_acc, d_hid), jnp.float32)
    zeros64 = jnp.zeros((n_acc, d_out), jnp.float32)

    deg_parts = _make_deg(n_acc, ep)(dst_p, zeros1)
    deg_t = jnp.transpose(deg_parts[:, :n])

    u1, dis = _tc_encode1(x, W1, deg_t)
    parts1 = _make_agg(n_acc, d_hid, ep)(u1, src_p, dst_p, zeros128)
    u2 = _tc_mid(parts1[0, :n], parts1[1, :n], u1, dis,
                 b1.reshape(1, d_hid), W2)

    parts2 = _make_agg(n_acc, d_out, ep)(u2, src_p, dst_p, zeros64)
    z = _tc_final(parts2[0, :n], parts2[1, :n], u2, dis,
                  b2.reshape(1, d_out))

    pf = _make_dec(d_out, ep_dec)(z, a_p, b_p)
    p3 = pf.reshape(ep_dec * 16 // 2048, 128, 16)
    s2 = _tc_reduce16(p3)
    return s2.reshape(-1)[:e_dec]


# speedup vs baseline: 12.2299x; 12.2299x over previous
import functools

import jax
import jax.numpy as jnp
from jax import lax
from jax.experimental import pallas as pl
from jax.experimental.pallas import tpu as pltpu
from jax.experimental.pallas import tpu_sc as plsc

NC = 2
NS = 16
NW = NC * NS
CH = 128
GARB = 2288

_MESH = dict(core_axis_name="c", subcore_axis_name="s")

_SC_TILING = pltpu.CompilerParams(use_tc_tiling_on_sc=False)


def _zero_fill_1d(ref, size):
    def b(i, carry):
        ref[pl.ds(i * 16, 16)] = jnp.zeros((16,), jnp.float32)
        return carry

    lax.fori_loop(0, size // 16, b, 0)


def _zero_fill_2d(ref, r, d):
    def b(i, carry):
        for j in range(d // 16):
            ref[i, pl.ds(16 * j, 16)] = jnp.zeros((16,), jnp.float32)
        return carry

    lax.fori_loop(0, r, b, 0)


def _pad_edges(idx_val, idx_tgt, n):
    e = idx_val.shape[0]
    ep = ((e + NW * CH - 1) // (NW * CH)) * (NW * CH)
    pad = ep - e
    ar = jnp.arange(pad, dtype=jnp.int32)
    val_p = jnp.concatenate([idx_val, ar % n])
    tgt_p = jnp.concatenate([idx_tgt, n + (ar % GARB)])
    return val_p, tgt_p, ep


def _make_deg(n_acc, ep):
    ew = ep // NW
    cpt = ew // CH
    rpt = n_acc // NS
    mesh = plsc.VectorSubcoreMesh(**_MESH)

    @functools.partial(
        pl.kernel,
        mesh=mesh,
        out_type=jax.ShapeDtypeStruct((NC * n_acc,), jnp.float32),
        scratch_types=[
            pltpu.VMEM((CH,), jnp.int32),
            pltpu.VMEM((CH,), jnp.float32),
            pltpu.VMEM((rpt,), jnp.float32),
            pltpu.VMEM_SHARED((n_acc,), jnp.float32),
        ],
    )
    def deg_k(dst_hbm, out_hbm, idx_d, ones_v, zbuf, deg_sh):
        c = lax.axis_index("c")
        s = lax.axis_index("s")
        wid = c * NS + s
        for j in range(CH // 16):
            ones_v[pl.ds(16 * j, 16)] = jnp.ones((16,), jnp.float32)
        r0 = s * rpt
        _zero_fill_1d(zbuf, rpt)
        pltpu.sync_copy(zbuf, deg_sh.at[pl.ds(r0, rpt)])
        plsc.subcore_barrier()

        def body(i, carry):
            base = wid * ew + i * CH
            pltpu.sync_copy(dst_hbm.at[pl.ds(base, CH)], idx_d)
            pltpu.sync_copy(ones_v, deg_sh.at[idx_d], add=True)
            return carry

        lax.fori_loop(0, cpt, body, 0)
        plsc.subcore_barrier()
        pltpu.sync_copy(deg_sh.at[pl.ds(r0, rpt)],
                        out_hbm.at[pl.ds(c * n_acc + r0, rpt)])

    return deg_k


def _make_agg(n_acc, d, ep, sc_tiling=False):
    ew = ep // NW
    cpt = ew // CH
    rpt = n_acc // NS
    mesh = plsc.VectorSubcoreMesh(**_MESH)

    @functools.partial(
        pl.kernel,
        mesh=mesh,
        compiler_params=_SC_TILING if sc_tiling else None,
        out_type=jax.ShapeDtypeStruct((NC * n_acc, d), jnp.float32),
        scratch_types=[
            pltpu.VMEM((CH,), jnp.int32),
            pltpu.VMEM((CH,), jnp.int32),
            pltpu.VMEM((CH, d), jnp.float32),
            pltpu.VMEM_SHARED((n_acc, d), jnp.float32),
            pltpu.SemaphoreType.DMA,
        ],
    )
    def agg_k(u_hbm, src_hbm, dst_hbm, out_hbm,
              idx_s, idx_d, rows, acc_sh, sem):
        c = lax.axis_index("c")
        s = lax.axis_index("s")
        wid = c * NS + s
        r0 = s * rpt
        _zero_fill_2d(rows, CH, d)
        for k in range(rpt // CH):
            pltpu.sync_copy(rows, acc_sh.at[pl.ds(r0 + k * CH, CH)])
        plsc.subcore_barrier()

        def body(i, carry):
            base = wid * ew + i * CH
            pltpu.sync_copy(src_hbm.at[pl.ds(base, CH)], idx_s)
            pltpu.sync_copy(dst_hbm.at[pl.ds(base, CH)], idx_d)
            pltpu.async_copy(u_hbm.at[idx_s], rows, sem).wait()
            pltpu.sync_copy(rows, acc_sh.at[idx_d], add=True)
            return carry

        lax.fori_loop(0, cpt, body, 0)
        plsc.subcore_barrier()
        pltpu.sync_copy(acc_sh.at[pl.ds(r0, rpt)],
                        out_hbm.at[pl.ds(c * n_acc + r0, rpt)])

    return agg_k


def _make_dec(d, ep):
    ew = ep // NW
    cpt = ew // CH
    mesh = plsc.VectorSubcoreMesh(**_MESH)

    @functools.partial(
        pl.kernel,
        mesh=mesh,
        compiler_params=_SC_TILING,
        out_type=jax.ShapeDtypeStruct((ep * 16,), jnp.float32),
        scratch_types=[
            pltpu.VMEM((CH,), jnp.int32),
            pltpu.VMEM((CH,), jnp.int32),
            pltpu.VMEM((CH, d), jnp.float32),
            pltpu.VMEM((CH, d), jnp.float32),
            pltpu.VMEM((CH * 16,), jnp.float32),
            pltpu.SemaphoreType.DMA,
            pltpu.SemaphoreType.DMA,
        ],
    )
    def dec_k(z_hbm, a_hbm, b_hbm, out_hbm,
              idx_a, idx_b, za, zb, part, sema, semb):
        c = lax.axis_index("c")
        s = lax.axis_index("s")
        wid = c * NS + s

        def body(i, carry):
            base = wid * ew + i * CH
            pltpu.sync_copy(a_hbm.at[pl.ds(base, CH)], idx_a)
            pltpu.sync_copy(b_hbm.at[pl.ds(base, CH)], idx_b)
            ca = pltpu.async_copy(z_hbm.at[idx_a], za, sema)
            cb = pltpu.async_copy(z_hbm.at[idx_b], zb, semb)
            ca.wait()
            cb.wait()

            def edge(e2, carry2):
                acc = za[e2, pl.ds(0, 16)] * zb[e2, pl.ds(0, 16)]
                for j in range(1, d // 16):
                    acc = acc + za[e2, pl.ds(16 * j, 16)] * zb[e2, pl.ds(16 * j, 16)]
                part[pl.ds(e2 * 16, 16)] = acc
                return carry2

            lax.fori_loop(0, CH, edge, 0)
            pltpu.sync_copy(part, out_hbm.at[pl.ds(base * 16, CH * 16)])
            return carry

        lax.fori_loop(0, cpt, body, 0)

    return dec_k


def _tc_encode1(x, W1, deg_t):
    n, d_hid = x.shape[0], W1.shape[1]

    def body(x_ref, w_ref, deg_ref, u1_ref, dis_ref):
        deg = deg_ref[:, 0:1] + deg_ref[:, 1:2] + 1.0
        dis = lax.rsqrt(deg)
        xw = jnp.dot(x_ref[...], w_ref[...],
                     preferred_element_type=jnp.float32,
                     precision=lax.Precision.HIGHEST)
        u1_ref[...] = xw * dis
        dis_ref[...] = dis

    return pl.pallas_call(
        body,
        out_shape=(jax.ShapeDtypeStruct((n, d_hid), jnp.float32),
                   jax.ShapeDtypeStruct((n, 1), jnp.float32)),
    )(x, W1, deg_t)


def _tc_mid(pa, pb, u1, dis, b1, W2):
    n = u1.shape[0]
    d_out = W2.shape[1]

    def body(pa_ref, pb_ref, u1_ref, dis_ref, b1_ref, w2_ref, u2_ref):
        acc = pa_ref[...] + pb_ref[...] + u1_ref[...]
        h = jnp.maximum(dis_ref[...] * acc + b1_ref[...], 0.0)
        hw = jnp.dot(h, w2_ref[...],
                     preferred_element_type=jnp.float32,
                     precision=lax.Precision.HIGHEST)
        u2_ref[...] = hw * dis_ref[...]

    return pl.pallas_call(
        body,
        out_shape=jax.ShapeDtypeStruct((n, d_out), jnp.float32),
    )(pa, pb, u1, dis, b1, W2)


def _tc_final(pa, pb, u2, dis, b2):
    n, d_out = u2.shape

    def body(pa_ref, pb_ref, u2_ref, dis_ref, b2_ref, z_ref):
        acc = pa_ref[...] + pb_ref[...] + u2_ref[...]
        z_ref[...] = dis_ref[...] * acc + b2_ref[...]

    return pl.pallas_call(
        body,
        out_shape=jax.ShapeDtypeStruct((n, d_out), jnp.float32),
    )(pa, pb, u2, dis, b2)


def _tc_reduce16(p2d, sel):
    m = p2d.shape[0]

    def body(p_ref, s_ref, o_ref):
        o_ref[...] = jnp.dot(p_ref[...], s_ref[...],
                             preferred_element_type=jnp.float32,
                             precision=lax.Precision.HIGHEST)

    return pl.pallas_call(
        body,
        out_shape=jax.ShapeDtypeStruct((m, 128), jnp.float32),
    )(p2d, sel)


@jax.jit
def kernel(x, edge_index, pos_edge_index, neg_edge_index, W1, b1, W2, b2):
    n = x.shape[0]
    d_hid = W1.shape[1]
    d_out = W2.shape[1]
    n_acc = n + GARB

    src, dst = edge_index[0], edge_index[1]
    src_p, dst_p, ep = _pad_edges(src, dst, n)

    ei = jnp.concatenate([pos_edge_index, neg_edge_index], axis=1)
    e_dec = ei.shape[1]
    a_p, b_p, ep_dec = _pad_edges(ei[0], ei[1], n)
    b_p = jnp.where(jnp.arange(ep_dec) < e_dec, b_p, b_p % n)

    deg_parts = _make_deg(n_acc, ep)(dst_p).reshape(NC, n_acc)
    deg_t = jnp.transpose(deg_parts[:, :n])

    u1, dis = _tc_encode1(x, W1, deg_t)
    parts1 = _make_agg(n_acc, d_hid, ep)(u1, src_p, dst_p)
    u2 = _tc_mid(parts1[:n], parts1[n_acc:n_acc + n], u1, dis,
                 b1.reshape(1, d_hid), W2)

    parts2 = _make_agg(n_acc, d_out, ep, sc_tiling=True)(u2, src_p, dst_p)
    z = _tc_final(parts2[:n], parts2[n_acc:n_acc + n], u2, dis,
                  b2.reshape(1, d_out))

    pf = _make_dec(d_out, ep_dec)(z, a_p, b_p)
    p2d = pf.reshape(ep_dec * 16 // 2048, 2048)
    sel = (jnp.arange(2048, dtype=jnp.int32)[:, None] // 16
           == jnp.arange(128, dtype=jnp.int32)[None, :]).astype(jnp.float32)
    s2 = _tc_reduce16(p2d, sel)
    return s2.reshape(-1)[:e_dec]


# speedup vs baseline: 15.6221x; 1.2774x over previous
import functools

import jax
import jax.numpy as jnp
from jax import lax
from jax.experimental import pallas as pl
from jax.experimental.pallas import tpu as pltpu
from jax.experimental.pallas import tpu_sc as plsc

NC = 2
NS = 16
NW = NC * NS
CH = 128
GARB = 240

_MESH = dict(core_axis_name="c", subcore_axis_name="s")

_SC_TILING = pltpu.CompilerParams(use_tc_tiling_on_sc=False)


def _zero_fill_1d(ref, size):
    def b(i, carry):
        ref[pl.ds(i * 16, 16)] = jnp.zeros((16,), jnp.float32)
        return carry

    lax.fori_loop(0, size // 16, b, 0)


def _zero_fill_2d(ref, r, d):
    def b(i, carry):
        for j in range(d // 16):
            ref[i, pl.ds(16 * j, 16)] = jnp.zeros((16,), jnp.float32)
        return carry

    lax.fori_loop(0, r, b, 0)


def _pad_edges(idx_val, idx_tgt, n):
    e = idx_val.shape[0]
    ep = ((e + NW * CH - 1) // (NW * CH)) * (NW * CH)
    pad = ep - e
    ar = jnp.arange(pad, dtype=jnp.int32)
    val_p = jnp.concatenate([idx_val, ar % n])
    tgt_p = jnp.concatenate([idx_tgt, n + (ar % GARB)])
    return val_p, tgt_p, ep


def _make_deg(n_acc, ep):
    ew = ep // NW
    cpt = ew // CH
    rpt = n_acc // NS
    mesh = plsc.VectorSubcoreMesh(**_MESH)

    @functools.partial(
        pl.kernel,
        mesh=mesh,
        out_type=jax.ShapeDtypeStruct((NC * n_acc,), jnp.float32),
        scratch_types=[
            pltpu.VMEM((CH,), jnp.int32),
            pltpu.VMEM((CH,), jnp.float32),
            pltpu.VMEM((rpt,), jnp.float32),
            pltpu.VMEM_SHARED((n_acc,), jnp.float32),
        ],
    )
    def deg_k(dst_hbm, out_hbm, idx_d, ones_v, zbuf, deg_sh):
        c = lax.axis_index("c")
        s = lax.axis_index("s")
        wid = c * NS + s
        for j in range(CH // 16):
            ones_v[pl.ds(16 * j, 16)] = jnp.ones((16,), jnp.float32)
        r0 = s * rpt
        _zero_fill_1d(zbuf, rpt)
        pltpu.sync_copy(zbuf, deg_sh.at[pl.ds(r0, rpt)])
        plsc.subcore_barrier()

        def body(i, carry):
            base = wid * ew + i * CH
            pltpu.sync_copy(dst_hbm.at[pl.ds(base, CH)], idx_d)
            pltpu.sync_copy(ones_v, deg_sh.at[idx_d], add=True)
            return carry

        lax.fori_loop(0, cpt, body, 0)
        plsc.subcore_barrier()
        pltpu.sync_copy(deg_sh.at[pl.ds(r0, rpt)],
                        out_hbm.at[pl.ds(c * n_acc + r0, rpt)])

    return deg_k


def _make_agg(n_acc, d, ep, sc_tiling=False):
    ew = ep // NW
    cpt = ew // CH
    rpt = n_acc // NS
    mesh = plsc.VectorSubcoreMesh(**_MESH)

    @functools.partial(
        pl.kernel,
        mesh=mesh,
        compiler_params=_SC_TILING if sc_tiling else None,
        out_type=jax.ShapeDtypeStruct((NC * n_acc, d), jnp.float32),
        scratch_types=[
            pltpu.VMEM((CH,), jnp.int32),
            pltpu.VMEM((CH,), jnp.int32),
            pltpu.VMEM((CH,), jnp.int32),
            pltpu.VMEM((CH,), jnp.int32),
            pltpu.VMEM((CH, d), jnp.float32),
            pltpu.VMEM((CH, d), jnp.float32),
            pltpu.VMEM_SHARED((n_acc, d), jnp.float32),
            pltpu.SemaphoreType.DMA,
            pltpu.SemaphoreType.DMA,
        ],
    )
    def agg_k(u_hbm, src_hbm, dst_hbm, out_hbm,
              idx_sa, idx_da, idx_sb, idx_db, rows_a, rows_b,
              acc_sh, sem_a, sem_b):
        c = lax.axis_index("c")
        s = lax.axis_index("s")
        wid = c * NS + s
        r0 = s * rpt
        _zero_fill_2d(rows_a, CH, d)
        for k in range(rpt // CH):
            pltpu.sync_copy(rows_a, acc_sh.at[pl.ds(r0 + k * CH, CH)])
        plsc.subcore_barrier()

        def body(i, carry):
            base_a = wid * ew + (2 * i) * CH
            base_b = base_a + CH
            pltpu.sync_copy(src_hbm.at[pl.ds(base_a, CH)], idx_sa)
            pltpu.sync_copy(dst_hbm.at[pl.ds(base_a, CH)], idx_da)
            ca = pltpu.async_copy(u_hbm.at[idx_sa], rows_a, sem_a)
            pltpu.sync_copy(src_hbm.at[pl.ds(base_b, CH)], idx_sb)
            pltpu.sync_copy(dst_hbm.at[pl.ds(base_b, CH)], idx_db)
            cb = pltpu.async_copy(u_hbm.at[idx_sb], rows_b, sem_b)
            ca.wait()
            pltpu.sync_copy(rows_a, acc_sh.at[idx_da], add=True)
            cb.wait()
            pltpu.sync_copy(rows_b, acc_sh.at[idx_db], add=True)
            return carry

        lax.fori_loop(0, cpt // 2, body, 0)
        if cpt % 2:
            base = wid * ew + (cpt - 1) * CH
            pltpu.sync_copy(src_hbm.at[pl.ds(base, CH)], idx_sa)
            pltpu.sync_copy(dst_hbm.at[pl.ds(base, CH)], idx_da)
            pltpu.async_copy(u_hbm.at[idx_sa], rows_a, sem_a).wait()
            pltpu.sync_copy(rows_a, acc_sh.at[idx_da], add=True)
        plsc.subcore_barrier()
        pltpu.sync_copy(acc_sh.at[pl.ds(r0, rpt)],
                        out_hbm.at[pl.ds(c * n_acc + r0, rpt)])

    return agg_k


def _make_dec(d, ep):
    ew = ep // NW
    cpt = ew // CH
    mesh = plsc.VectorSubcoreMesh(**_MESH)

    @functools.partial(
        pl.kernel,
        mesh=mesh,
        compiler_params=_SC_TILING,
        out_type=jax.ShapeDtypeStruct((ep * 16,), jnp.float32),
        scratch_types=[
            pltpu.VMEM((CH,), jnp.int32),
            pltpu.VMEM((CH,), jnp.int32),
            pltpu.VMEM((CH,), jnp.int32),
            pltpu.VMEM((CH,), jnp.int32),
            pltpu.VMEM((CH, d), jnp.float32),
            pltpu.VMEM((CH, d), jnp.float32),
            pltpu.VMEM((CH, d), jnp.float32),
            pltpu.VMEM((CH, d), jnp.float32),
            pltpu.VMEM((CH * 16,), jnp.float32),
            pltpu.SemaphoreType.DMA,
            pltpu.SemaphoreType.DMA,
            pltpu.SemaphoreType.DMA,
            pltpu.SemaphoreType.DMA,
        ],
    )
    def dec_k(z_hbm, a_hbm, b_hbm, out_hbm,
              idx_a1, idx_b1, idx_a2, idx_b2,
              za1, zb1, za2, zb2, part, sa1, sb1, sa2, sb2):
        c = lax.axis_index("c")
        s = lax.axis_index("s")
        wid = c * NS + s

        def compute(za, zb, base):
            def edge(e2, carry2):
                acc = za[e2, pl.ds(0, 16)] * zb[e2, pl.ds(0, 16)]
                for j in range(1, d // 16):
                    acc = acc + za[e2, pl.ds(16 * j, 16)] * zb[e2, pl.ds(16 * j, 16)]
                part[pl.ds(e2 * 16, 16)] = acc
                return carry2

            lax.fori_loop(0, CH, edge, 0)
            pltpu.sync_copy(part, out_hbm.at[pl.ds(base * 16, CH * 16)])

        def body(i, carry):
            base_a = wid * ew + (2 * i) * CH
            base_b = base_a + CH
            pltpu.sync_copy(a_hbm.at[pl.ds(base_a, CH)], idx_a1)
            pltpu.sync_copy(b_hbm.at[pl.ds(base_a, CH)], idx_b1)
            ca1 = pltpu.async_copy(z_hbm.at[idx_a1], za1, sa1)
            cb1 = pltpu.async_copy(z_hbm.at[idx_b1], zb1, sb1)
            pltpu.sync_copy(a_hbm.at[pl.ds(base_b, CH)], idx_a2)
            pltpu.sync_copy(b_hbm.at[pl.ds(base_b, CH)], idx_b2)
            ca2 = pltpu.async_copy(z_hbm.at[idx_a2], za2, sa2)
            cb2 = pltpu.async_copy(z_hbm.at[idx_b2], zb2, sb2)
            ca1.wait()
            cb1.wait()
            compute(za1, zb1, base_a)
            ca2.wait()
            cb2.wait()
            compute(za2, zb2, base_b)
            return carry

        lax.fori_loop(0, cpt // 2, body, 0)
        if cpt % 2:
            base = wid * ew + (cpt - 1) * CH
            pltpu.sync_copy(a_hbm.at[pl.ds(base, CH)], idx_a1)
            pltpu.sync_copy(b_hbm.at[pl.ds(base, CH)], idx_b1)
            ca1 = pltpu.async_copy(z_hbm.at[idx_a1], za1, sa1)
            cb1 = pltpu.async_copy(z_hbm.at[idx_b1], zb1, sb1)
            ca1.wait()
            cb1.wait()
            compute(za1, zb1, base)

    return dec_k


def _tc_encode1(x, W1, deg_t):
    n, d_hid = x.shape[0], W1.shape[1]

    def body(x_ref, w_ref, deg_ref, u1_ref, dis_ref):
        deg = deg_ref[:, 0:1] + deg_ref[:, 1:2] + 1.0
        dis = lax.rsqrt(deg)
        xw = jnp.dot(x_ref[...], w_ref[...],
                     preferred_element_type=jnp.float32,
                     precision=lax.Precision.HIGHEST)
        u1_ref[...] = xw * dis
        dis_ref[...] = dis

    return pl.pallas_call(
        body,
        out_shape=(jax.ShapeDtypeStruct((n, d_hid), jnp.float32),
                   jax.ShapeDtypeStruct((n, 1), jnp.float32)),
    )(x, W1, deg_t)


def _tc_mid(pa, pb, u1, dis, b1, W2):
    n = u1.shape[0]
    d_out = W2.shape[1]

    def body(pa_ref, pb_ref, u1_ref, dis_ref, b1_ref, w2_ref, u2_ref):
        acc = pa_ref[...] + pb_ref[...] + u1_ref[...]
        h = jnp.maximum(dis_ref[...] * acc + b1_ref[...], 0.0)
        hw = jnp.dot(h, w2_ref[...],
                     preferred_element_type=jnp.float32,
                     precision=lax.Precision.HIGHEST)
        u2_ref[...] = hw * dis_ref[...]

    return pl.pallas_call(
        body,
        out_shape=jax.ShapeDtypeStruct((n, d_out), jnp.float32),
    )(pa, pb, u1, dis, b1, W2)


def _tc_final(pa, pb, u2, dis, b2):
    n, d_out = u2.shape

    def body(pa_ref, pb_ref, u2_ref, dis_ref, b2_ref, z_ref):
        acc = pa_ref[...] + pb_ref[...] + u2_ref[...]
        z_ref[...] = dis_ref[...] * acc + b2_ref[...]

    return pl.pallas_call(
        body,
        out_shape=jax.ShapeDtypeStruct((n, d_out), jnp.float32),
    )(pa, pb, u2, dis, b2)


def _tc_reduce16(p2d, sel):
    m = p2d.shape[0]

    def body(p_ref, s_ref, o_ref):
        o_ref[...] = jnp.dot(p_ref[...], s_ref[...],
                             preferred_element_type=jnp.float32,
                             precision=lax.Precision.HIGHEST)

    return pl.pallas_call(
        body,
        out_shape=jax.ShapeDtypeStruct((m, 128), jnp.float32),
    )(p2d, sel)


@jax.jit
def kernel(x, edge_index, pos_edge_index, neg_edge_index, W1, b1, W2, b2):
    n = x.shape[0]
    d_hid = W1.shape[1]
    d_out = W2.shape[1]
    n_acc = n + GARB

    src, dst = edge_index[0], edge_index[1]
    src_p, dst_p, ep = _pad_edges(src, dst, n)

    ei = jnp.concatenate([pos_edge_index, neg_edge_index], axis=1)
    e_dec = ei.shape[1]
    a_p, b_p, ep_dec = _pad_edges(ei[0], ei[1], n)
    b_p = jnp.where(jnp.arange(ep_dec) < e_dec, b_p, b_p % n)

    deg_parts = _make_deg(n_acc, ep)(dst_p).reshape(NC, n_acc)
    deg_t = jnp.transpose(deg_parts[:, :n])

    u1, dis = _tc_encode1(x, W1, deg_t)
    parts1 = _make_agg(n_acc, d_hid, ep)(u1, src_p, dst_p)
    u2 = _tc_mid(parts1[:n], parts1[n_acc:n_acc + n], u1, dis,
                 b1.reshape(1, d_hid), W2)

    parts2 = _make_agg(n_acc, d_out, ep, sc_tiling=True)(u2, src_p, dst_p)
    z = _tc_final(parts2[:n], parts2[n_acc:n_acc + n], u2, dis,
                  b2.reshape(1, d_out))

    pf = _make_dec(d_out, ep_dec)(z, a_p, b_p)
    p2d = pf.reshape(ep_dec * 16 // 2048, 2048)
    sel = (jnp.arange(2048, dtype=jnp.int32)[:, None] // 16
           == jnp.arange(128, dtype=jnp.int32)[None, :]).astype(jnp.float32)
    s2 = _tc_reduce16(p2d, sel)
    return s2.reshape(-1)[:e_dec]


# speedup vs baseline: 16.0231x; 1.0257x over previous
import functools

import jax
import jax.numpy as jnp
from jax import lax
from jax.experimental import pallas as pl
from jax.experimental.pallas import tpu as pltpu
from jax.experimental.pallas import tpu_sc as plsc

NC = 2
NS = 16
NW = NC * NS
CH = 128
GARB = 240

_MESH = dict(core_axis_name="c", subcore_axis_name="s")

_SC_TILING = pltpu.CompilerParams(use_tc_tiling_on_sc=False)


def _zero_fill_1d(ref, size):
    def b(i, carry):
        ref[pl.ds(i * 16, 16)] = jnp.zeros((16,), jnp.float32)
        return carry

    lax.fori_loop(0, size // 16, b, 0)


def _zero_fill_2d(ref, r, d):
    def b(i, carry):
        for j in range(d // 16):
            ref[i, pl.ds(16 * j, 16)] = jnp.zeros((16,), jnp.float32)
        return carry

    lax.fori_loop(0, r, b, 0)


def _pad_edges(idx_val, idx_tgt, n):
    e = idx_val.shape[0]
    ep = ((e + NW * 8 - 1) // (NW * 8)) * (NW * 8)
    pad = ep - e
    if pad == 0:
        return idx_val, idx_tgt, ep
    ar = jnp.arange(pad, dtype=jnp.int32)
    val_p = jnp.concatenate([idx_val, ar % n])
    tgt_p = jnp.concatenate([idx_tgt, n + (ar % GARB)])
    return val_p, tgt_p, ep


def _make_deg(n_acc, ep):
    ew = ep // NW
    cpt = ew // CH
    tail = ew % CH
    rpt = n_acc // NS
    mesh = plsc.VectorSubcoreMesh(**_MESH)

    @functools.partial(
        pl.kernel,
        mesh=mesh,
        out_type=jax.ShapeDtypeStruct((NC * n_acc,), jnp.float32),
        scratch_types=[
            pltpu.VMEM((CH,), jnp.int32),
            pltpu.VMEM((CH,), jnp.int32),
            pltpu.VMEM((CH,), jnp.float32),
            pltpu.VMEM((rpt,), jnp.float32),
            pltpu.VMEM_SHARED((n_acc,), jnp.float32),
        ],
    )
    def deg_k(dst_hbm, out_hbm, idx_a, idx_b, ones_v, zbuf, deg_sh):
        c = lax.axis_index("c")
        s = lax.axis_index("s")
        wid = c * NS + s
        for j in range(CH // 16):
            ones_v[pl.ds(16 * j, 16)] = jnp.ones((16,), jnp.float32)
        if CH % 16:
            ones_v[pl.ds(CH - 16, 16)] = jnp.ones((16,), jnp.float32)
        r0 = s * rpt
        _zero_fill_1d(zbuf, rpt)
        pltpu.sync_copy(zbuf, deg_sh.at[pl.ds(r0, rpt)])
        plsc.subcore_barrier()

        def body(i, carry):
            base_a = wid * ew + (2 * i) * CH
            pltpu.sync_copy(dst_hbm.at[pl.ds(base_a, CH)], idx_a)
            pltpu.sync_copy(dst_hbm.at[pl.ds(base_a + CH, CH)], idx_b)
            pltpu.sync_copy(ones_v, deg_sh.at[idx_a], add=True)
            pltpu.sync_copy(ones_v, deg_sh.at[idx_b], add=True)
            return carry

        lax.fori_loop(0, cpt // 2, body, 0)
        if cpt % 2:
            base = wid * ew + (cpt - 1) * CH
            pltpu.sync_copy(dst_hbm.at[pl.ds(base, CH)], idx_a)
            pltpu.sync_copy(ones_v, deg_sh.at[idx_a], add=True)
        if tail:
            base = wid * ew + cpt * CH
            for j in range(CH // 16):
                garb = 16 * j + jnp.arange(16, dtype=jnp.int32)
                idx_a[pl.ds(16 * j, 16)] = (n_acc - GARB) + garb % GARB
            pltpu.sync_copy(dst_hbm.at[pl.ds(base, tail)],
                            idx_a.at[pl.ds(0, tail)])
            pltpu.sync_copy(ones_v, deg_sh.at[idx_a], add=True)
        plsc.subcore_barrier()
        pltpu.sync_copy(deg_sh.at[pl.ds(r0, rpt)],
                        out_hbm.at[pl.ds(c * n_acc + r0, rpt)])

    return deg_k


def _make_agg(n_acc, d, ep, sc_tiling=False):
    ew = ep // NW
    cpt = ew // CH
    tail = ew % CH
    rpt = n_acc // NS
    mesh = plsc.VectorSubcoreMesh(**_MESH)

    @functools.partial(
        pl.kernel,
        mesh=mesh,
        compiler_params=_SC_TILING if sc_tiling else None,
        out_type=jax.ShapeDtypeStruct((NC * n_acc, d), jnp.float32),
        scratch_types=[
            pltpu.VMEM((CH,), jnp.int32),
            pltpu.VMEM((CH,), jnp.int32),
            pltpu.VMEM((CH,), jnp.int32),
            pltpu.VMEM((CH,), jnp.int32),
            pltpu.VMEM((CH, d), jnp.float32),
            pltpu.VMEM((CH, d), jnp.float32),
            pltpu.VMEM_SHARED((n_acc, d), jnp.float32),
            pltpu.SemaphoreType.DMA,
            pltpu.SemaphoreType.DMA,
        ],
    )
    def agg_k(u_hbm, src_hbm, dst_hbm, out_hbm,
              idx_sa, idx_da, idx_sb, idx_db, rows_a, rows_b,
              acc_sh, sem_a, sem_b):
        c = lax.axis_index("c")
        s = lax.axis_index("s")
        wid = c * NS + s
        r0 = s * rpt
        _zero_fill_2d(rows_a, CH, d)
        for k in range(rpt // CH):
            pltpu.sync_copy(rows_a, acc_sh.at[pl.ds(r0 + k * CH, CH)])
        if rpt % CH:
            pltpu.sync_copy(rows_a.at[pl.ds(0, rpt % CH)],
                            acc_sh.at[pl.ds(r0 + (rpt // CH) * CH, rpt % CH)])
        plsc.subcore_barrier()

        def body(i, carry):
            base_a = wid * ew + (2 * i) * CH
            base_b = base_a + CH
            pltpu.sync_copy(src_hbm.at[pl.ds(base_a, CH)], idx_sa)
            pltpu.sync_copy(dst_hbm.at[pl.ds(base_a, CH)], idx_da)
            ca = pltpu.async_copy(u_hbm.at[idx_sa], rows_a, sem_a)
            pltpu.sync_copy(src_hbm.at[pl.ds(base_b, CH)], idx_sb)
            pltpu.sync_copy(dst_hbm.at[pl.ds(base_b, CH)], idx_db)
            cb = pltpu.async_copy(u_hbm.at[idx_sb], rows_b, sem_b)
            ca.wait()
            pltpu.sync_copy(rows_a, acc_sh.at[idx_da], add=True)
            cb.wait()
            pltpu.sync_copy(rows_b, acc_sh.at[idx_db], add=True)
            return carry

        lax.fori_loop(0, cpt // 2, body, 0)
        if cpt % 2:
            base = wid * ew + (cpt - 1) * CH
            pltpu.sync_copy(src_hbm.at[pl.ds(base, CH)], idx_sa)
            pltpu.sync_copy(dst_hbm.at[pl.ds(base, CH)], idx_da)
            pltpu.async_copy(u_hbm.at[idx_sa], rows_a, sem_a).wait()
            pltpu.sync_copy(rows_a, acc_sh.at[idx_da], add=True)
        if tail:
            base = wid * ew + cpt * CH
            for j in range(CH // 16):
                garb = 16 * j + jnp.arange(16, dtype=jnp.int32)
                idx_sa[pl.ds(16 * j, 16)] = garb
                idx_da[pl.ds(16 * j, 16)] = (n_acc - GARB) + garb % GARB
            pltpu.sync_copy(src_hbm.at[pl.ds(base, tail)],
                            idx_sa.at[pl.ds(0, tail)])
            pltpu.sync_copy(dst_hbm.at[pl.ds(base, tail)],
                            idx_da.at[pl.ds(0, tail)])
            pltpu.async_copy(u_hbm.at[idx_sa], rows_a, sem_a).wait()
            pltpu.sync_copy(rows_a, acc_sh.at[idx_da], add=True)
        plsc.subcore_barrier()
        pltpu.sync_copy(acc_sh.at[pl.ds(r0, rpt)],
                        out_hbm.at[pl.ds(c * n_acc + r0, rpt)])

    return agg_k


def _make_dec(d, ep):
    ew = ep // NW
    cpt = ew // CH
    tail = ew % CH
    mesh = plsc.VectorSubcoreMesh(**_MESH)

    @functools.partial(
        pl.kernel,
        mesh=mesh,
        compiler_params=_SC_TILING,
        out_type=jax.ShapeDtypeStruct((ep * 16,), jnp.float32),
        scratch_types=[
            pltpu.VMEM((CH,), jnp.int32),
            pltpu.VMEM((CH,), jnp.int32),
            pltpu.VMEM((CH,), jnp.int32),
            pltpu.VMEM((CH,), jnp.int32),
            pltpu.VMEM((CH, d), jnp.float32),
            pltpu.VMEM((CH, d), jnp.float32),
            pltpu.VMEM((CH, d), jnp.float32),
            pltpu.VMEM((CH, d), jnp.float32),
            pltpu.VMEM((CH * 16,), jnp.float32),
            pltpu.SemaphoreType.DMA,
            pltpu.SemaphoreType.DMA,
            pltpu.SemaphoreType.DMA,
            pltpu.SemaphoreType.DMA,
        ],
    )
    def dec_k(z_hbm, a_hbm, b_hbm, out_hbm,
              idx_a1, idx_b1, idx_a2, idx_b2,
              za1, zb1, za2, zb2, part, sa1, sb1, sa2, sb2):
        c = lax.axis_index("c")
        s = lax.axis_index("s")
        wid = c * NS + s

        UNR = 4
        assert CH % UNR == 0

        def fill_part(za, zb, m):
            def edge(q, carry2):
                e0 = q * UNR
                for u in range(UNR):
                    e2 = e0 + u
                    acc = za[e2, pl.ds(0, 16)] * zb[e2, pl.ds(0, 16)]
                    for j in range(1, d // 16):
                        acc = acc + za[e2, pl.ds(16 * j, 16)] * zb[e2, pl.ds(16 * j, 16)]
                    part[pl.ds(e2 * 16, 16)] = acc
                return carry2

            lax.fori_loop(0, m // UNR, edge, 0)

        def compute(za, zb, base):
            fill_part(za, zb, CH)
            pltpu.sync_copy(part, out_hbm.at[pl.ds(base * 16, CH * 16)])

        def body(i, carry):
            base_a = wid * ew + (2 * i) * CH
            base_b = base_a + CH
            pltpu.sync_copy(a_hbm.at[pl.ds(base_a, CH)], idx_a1)
            pltpu.sync_copy(b_hbm.at[pl.ds(base_a, CH)], idx_b1)
            ca1 = pltpu.async_copy(z_hbm.at[idx_a1], za1, sa1)
            cb1 = pltpu.async_copy(z_hbm.at[idx_b1], zb1, sb1)
            pltpu.sync_copy(a_hbm.at[pl.ds(base_b, CH)], idx_a2)
            pltpu.sync_copy(b_hbm.at[pl.ds(base_b, CH)], idx_b2)
            ca2 = pltpu.async_copy(z_hbm.at[idx_a2], za2, sa2)
            cb2 = pltpu.async_copy(z_hbm.at[idx_b2], zb2, sb2)
            ca1.wait()
            cb1.wait()
            compute(za1, zb1, base_a)
            ca2.wait()
            cb2.wait()
            compute(za2, zb2, base_b)
            return carry

        lax.fori_loop(0, cpt // 2, body, 0)
        if cpt % 2:
            base = wid * ew + (cpt - 1) * CH
            pltpu.sync_copy(a_hbm.at[pl.ds(base, CH)], idx_a1)
            pltpu.sync_copy(b_hbm.at[pl.ds(base, CH)], idx_b1)
            ca1 = pltpu.async_copy(z_hbm.at[idx_a1], za1, sa1)
            cb1 = pltpu.async_copy(z_hbm.at[idx_b1], zb1, sb1)
            ca1.wait()
            cb1.wait()
            compute(za1, zb1, base)
        if tail:
            base = wid * ew + cpt * CH
            for j in range(CH // 16):
                garb = 16 * j + jnp.arange(16, dtype=jnp.int32)
                idx_a1[pl.ds(16 * j, 16)] = garb
                idx_b1[pl.ds(16 * j, 16)] = garb
            pltpu.sync_copy(a_hbm.at[pl.ds(base, tail)],
                            idx_a1.at[pl.ds(0, tail)])
            pltpu.sync_copy(b_hbm.at[pl.ds(base, tail)],
                            idx_b1.at[pl.ds(0, tail)])
            ca1 = pltpu.async_copy(z_hbm.at[idx_a1], za1, sa1)
            cb1 = pltpu.async_copy(z_hbm.at[idx_b1], zb1, sb1)
            ca1.wait()
            cb1.wait()
            fill_part(za1, zb1, tail)
            pltpu.sync_copy(part.at[pl.ds(0, tail * 16)],
                            out_hbm.at[pl.ds(base * 16, tail * 16)])

    return dec_k


def _tc_encode1(x, W1, deg_t):
    n, d_hid = x.shape[0], W1.shape[1]

    def body(x_ref, w_ref, deg_ref, u1_ref, dis_ref):
        deg = deg_ref[:, 0:1] + deg_ref[:, 1:2] + 1.0
        dis = lax.rsqrt(deg)
        xw = jnp.dot(x_ref[...], w_ref[...],
                     preferred_element_type=jnp.float32,
                     precision=lax.Precision.HIGHEST)
        u1_ref[...] = xw * dis
        dis_ref[...] = dis

    return pl.pallas_call(
        body,
        out_shape=(jax.ShapeDtypeStruct((n, d_hid), jnp.float32),
                   jax.ShapeDtypeStruct((n, 1), jnp.float32)),
    )(x, W1, deg_t)


def _tc_mid(pa, pb, u1, dis, b1, W2):
    n = u1.shape[0]
    d_out = W2.shape[1]

    def body(pa_ref, pb_ref, u1_ref, dis_ref, b1_ref, w2_ref, u2_ref):
        acc = pa_ref[...] + pb_ref[...] + u1_ref[...]
        h = jnp.maximum(dis_ref[...] * acc + b1_ref[...], 0.0)
        hw = jnp.dot(h, w2_ref[...],
                     preferred_element_type=jnp.float32,
                     precision=lax.Precision.HIGHEST)
        u2_ref[...] = hw * dis_ref[...]

    return pl.pallas_call(
        body,
        out_shape=jax.ShapeDtypeStruct((n, d_out), jnp.float32),
    )(pa, pb, u1, dis, b1, W2)


def _tc_final(pa, pb, u2, dis, b2):
    n, d_out = u2.shape

    def body(pa_ref, pb_ref, u2_ref, dis_ref, b2_ref, z_ref):
        acc = pa_ref[...] + pb_ref[...] + u2_ref[...]
        z_ref[...] = dis_ref[...] * acc + b2_ref[...]

    return pl.pallas_call(
        body,
        out_shape=jax.ShapeDtypeStruct((n, d_out), jnp.float32),
    )(pa, pb, u2, dis, b2)


def _tc_reduce16(p2d, sel):
    m = p2d.shape[0]

    def body(p_ref, s_ref, o_ref):
        o_ref[...] = jnp.dot(p_ref[...], s_ref[...],
                             preferred_element_type=jnp.float32,
                             precision=lax.Precision.HIGHEST)

    return pl.pallas_call(
        body,
        out_shape=jax.ShapeDtypeStruct((m, 128), jnp.float32),
    )(p2d, sel)


@jax.jit
def kernel(x, edge_index, pos_edge_index, neg_edge_index, W1, b1, W2, b2):
    n = x.shape[0]
    d_hid = W1.shape[1]
    d_out = W2.shape[1]
    n_acc = n + GARB

    src, dst = edge_index[0], edge_index[1]
    src_p, dst_p, ep = _pad_edges(src, dst, n)

    ei = jnp.concatenate([pos_edge_index, neg_edge_index], axis=1)
    e_dec = ei.shape[1]
    a_p, b_p, ep_dec = _pad_edges(ei[0], ei[1], n)
    if ep_dec != e_dec:
        b_p = jnp.where(jnp.arange(ep_dec) < e_dec, b_p, b_p % n)

    deg_parts = _make_deg(n_acc, ep)(dst_p).reshape(NC, n_acc)
    deg_t = jnp.transpose(deg_parts[:, :n])

    u1, dis = _tc_encode1(x, W1, deg_t)
    parts1 = _make_agg(n_acc, d_hid, ep)(u1, src_p, dst_p)
    u2 = _tc_mid(parts1[:n], parts1[n_acc:n_acc + n], u1, dis,
                 b1.reshape(1, d_hid), W2)

    parts2 = _make_agg(n_acc, d_out, ep, sc_tiling=True)(u2, src_p, dst_p)
    z = _tc_final(parts2[:n], parts2[n_acc:n_acc + n], u2, dis,
                  b2.reshape(1, d_out))

    pf = _make_dec(d_out, ep_dec)(z, a_p, b_p)
    p2d = pf.reshape(ep_dec * 16 // 2048, 2048)
    sel = (jnp.arange(2048, dtype=jnp.int32)[:, None] // 16
           == jnp.arange(128, dtype=jnp.int32)[None, :]).astype(jnp.float32)
    s2 = _tc_reduce16(p2d, sel)
    return s2.reshape(-1)[:e_dec]
